# baseline (speedup 1.0000x reference)
import functools

import jax
import jax.numpy as jnp
from jax import lax
from jax.experimental import pallas as pl
from jax.experimental.pallas import tpu as pltpu
from jax.experimental.pallas import tpu_sc as plsc

_N = 10000
_NP = 10240
_D = 128
_E = 320000
_NC, _NS = 2, 16
_NW = _NC * _NS
_EPT = 10240
_CHUNK = 128
_NCHUNKS = _EPT // _CHUNK
_RPS = _NP // _NS
_PAD_DST = 10016
_DW = 16


def _sc_mesh():
    return plsc.VectorSubcoreMesh(core_axis_name="c", subcore_axis_name="s")


def _sc_degree(dstp):
    @functools.partial(
        pl.kernel,
        mesh=_sc_mesh(),
        out_type=jax.ShapeDtypeStruct((_NC, _NP), jnp.float32),
        scratch_types=[
            pltpu.VMEM((_NCHUNKS, _CHUNK), jnp.int32),
            pltpu.VMEM((_CHUNK,), jnp.float32),
            pltpu.VMEM_SHARED((_NP,), jnp.float32),
        ],
    )
    def k(dst_hbm, out_hbm, dst_v, ones_v, deg_sh):
        cid = lax.axis_index("c")
        sid = lax.axis_index("s")
        wid = sid * _NC + cid
        pltpu.sync_copy(dst_hbm.at[pl.ds(wid * _NCHUNKS, _NCHUNKS)], dst_v)

        ones16 = jnp.ones((16,), jnp.float32)
        zeros16 = jnp.zeros((16,), jnp.float32)

        def _z(r, c):
            ones_v[pl.ds(r * 16, 16)] = zeros16
            return c

        lax.fori_loop(0, _CHUNK // 16, _z, 0)
        base = sid * _RPS
        for t in range(_RPS // _CHUNK):
            pltpu.sync_copy(ones_v, deg_sh.at[pl.ds(base + t * _CHUNK, _CHUNK)])

        def _o(r, c):
            ones_v[pl.ds(r * 16, 16)] = ones16
            return c

        lax.fori_loop(0, _CHUNK // 16, _o, 0)
        plsc.subcore_barrier()

        def _step(j, c):
            pltpu.sync_copy(ones_v, deg_sh.at[dst_v.at[j]], add=True)
            return c

        lax.fori_loop(0, _NCHUNKS, _step, 0)
        plsc.subcore_barrier()
        pltpu.sync_copy(deg_sh.at[pl.ds(base, _RPS)],
                        out_hbm.at[cid, pl.ds(base, _RPS)])

    return k(dstp)


_TOTCH = _NW * _NCHUNKS
_CH0 = 40
_CH1 = (_TOTCH - _CH0 * _NS) // _NS
_CHMAX = max(_CH0, _CH1)


def _sc_scatter(h_tab, srcp, dstp):
    @functools.partial(
        pl.kernel,
        mesh=_sc_mesh(),
        out_type=jax.ShapeDtypeStruct((_NC, _NP, _D), jnp.float32),
        scratch_types=[
            pltpu.VMEM((_CHMAX, _CHUNK), jnp.int32),
            pltpu.VMEM((_CHMAX, _CHUNK), jnp.int32),
            pltpu.VMEM((_CHUNK, _D), jnp.float32),
            pltpu.SemaphoreType.DMA,
            pltpu.VMEM_SHARED((_NP, _D), jnp.float32),
        ],
    )
    def k(h_hbm, src_hbm, dst_hbm, out_hbm, src_v, dst_v, rows_v, sem,
          acc_sh):
        cid = lax.axis_index("c")
        sid = lax.axis_index("s")
        chbase = jnp.where(cid == 0, sid * _CH0, _CH0 * _NS + sid * _CH1)
        nch = jnp.where(cid == 0, _CH0, _CH1)
        pltpu.sync_copy(src_hbm.at[pl.ds(chbase, _CHMAX)], src_v)
        pltpu.sync_copy(dst_hbm.at[pl.ds(chbase, _CHMAX)], dst_v)

        zeros16 = jnp.zeros((16,), jnp.float32)

        def _z(r, c):
            for q in range(_D // 16):
                rows_v[r, pl.ds(q * 16, 16)] = zeros16
            return c

        lax.fori_loop(0, _CHUNK, _z, 0)

        base = sid * _RPS
        for t in range(_RPS // _CHUNK):
            pltpu.sync_copy(rows_v, acc_sh.at[pl.ds(base + t * _CHUNK, _CHUNK)])
        plsc.subcore_barrier()

        def _step(j, c):
            pltpu.async_copy(h_hbm.at[src_v.at[j]], rows_v, sem).wait()
            pltpu.sync_copy(rows_v, acc_sh.at[dst_v.at[j]], add=True)
            return c

        lax.fori_loop(0, nch, _step, 0)
        plsc.subcore_barrier()
        pltpu.sync_copy(acc_sh.at[pl.ds(base, _RPS)],
                        out_hbm.at[cid, pl.ds(base, _RPS)])

    return k(h_tab, srcp, dstp)


_R = 2048
_G = _NP // _R


def _dot(a, b):
    return jnp.dot(a, b, preferred_element_type=jnp.float32)


def _row_spec(width=_D):
    return pl.BlockSpec((_R, width), lambda i: (i, 0))


def _full_spec(shape):
    nd = len(shape)
    return pl.BlockSpec(shape, lambda i: (0,) * nd)


def _tc0(xp, gxp, w1a, w1b, b1, wdr, bdr, wg1):
    def body(x_r, gx_r, w1a_r, w1b_r, b1_r, wdr_r, bdr_r, wg1_r,
             z_r, z0_r, h1_r):
        gg = gx_r[...]
        z = jnp.maximum(
            _dot(x_r[...], w1a_r[...]) + _dot(gg, w1b_r[...]) + b1_r[...], 0.0)
        z_r[...] = z
        z0_r[...] = _dot(z, wdr_r[...]) + bdr_r[...]
        h1_r[...] = _dot(z + gg, wg1_r[...])

    out = jax.ShapeDtypeStruct((_NP, _D), jnp.float32)
    return pl.pallas_call(
        body,
        grid=(_G,),
        in_specs=[_row_spec(), _row_spec(),
                  _full_spec((_D, _D)), _full_spec((_D, _D)),
                  _full_spec((1, _D)),
                  _full_spec((_D, _D)), _full_spec((1, _D)),
                  _full_spec((_D, _D))],
        out_specs=[_row_spec(), _row_spec(), _row_spec()],
        out_shape=[out, out, out],
    )(xp, gxp, w1a, w1b, b1, wdr, bdr, wg1)


def _tc1(d0, d1, h1):
    def body(d0_r, d1_r, h1_r, hp_r, dinv_r):
        deg = d0_r[...] + d1_r[...] + 1.0
        dinv = lax.rsqrt(deg)
        dinv_r[...] = dinv
        hp_r[...] = dinv * h1_r[...]

    return pl.pallas_call(
        body,
        grid=(_G,),
        in_specs=[_row_spec(1), _row_spec(1), _row_spec()],
        out_specs=[_row_spec(), _row_spec(1)],
        out_shape=[jax.ShapeDtypeStruct((_NP, _D), jnp.float32),
                   jax.ShapeDtypeStruct((_NP, 1), jnp.float32)],
    )(d0, d1, h1)


def _tc2(s0, s1, hp, dinv, bg1, wg2):
    def body(s0_r, s1_r, hp_r, dinv_r, bg1_r, wg2_r, z1_r, h2p_r):
        dv = dinv_r[...]
        z1 = jnp.maximum(dv * (s0_r[...] + s1_r[...] + hp_r[...]) + bg1_r[...],
                         0.0)
        z1_r[...] = z1
        h2p_r[...] = dv * _dot(z1, wg2_r[...])

    out = jax.ShapeDtypeStruct((_NP, _D), jnp.float32)
    return pl.pallas_call(
        body,
        grid=(_G,),
        in_specs=[_row_spec(), _row_spec(), _row_spec(), _row_spec(1),
                  _full_spec((1, _D)), _full_spec((_D, _D))],
        out_specs=[_row_spec(), _row_spec()],
        out_shape=[out, out],
    )(s0, s1, hp, dinv, bg1, wg2)


def _tc3(t0, t1, h2p, dinv, bg2, z, z1, z0, w2a, w2b, w2c, b2, w3, b3, wo, bo):
    def body(t0_r, t1_r, h2p_r, dinv_r, bg2_r, z_r, z1_r, z0_r,
             w2a_r, w2b_r, w2c_r, b2_r, w3_r, b3_r, wo_r, bo_r, o_r):
        dv = dinv_r[...]
        z2 = jnp.maximum(
            dv * (t0_r[...] + t1_r[...] + h2p_r[...]) + bg2_r[...], 0.0)
        z3 = jnp.maximum(
            _dot(z_r[...], w2a_r[...]) + _dot(z1_r[...], w2b_r[...])
            + _dot(z2, w2c_r[...]) + b2_r[...], 0.0)
        z4 = jnp.maximum(_dot(z3 + z0_r[...], w3_r[...]) + b3_r[...], 0.0)
        o_r[...] = _dot(z4, wo_r[...]) + bo_r[...]

    return pl.pallas_call(
        body,
        grid=(_G,),
        in_specs=[_row_spec(), _row_spec(), _row_spec(), _row_spec(1),
                  _full_spec((1, _D)),
                  _row_spec(), _row_spec(), _row_spec(),
                  _full_spec((_D, _D)), _full_spec((_D, _D)),
                  _full_spec((_D, _D)), _full_spec((1, _D)),
                  _full_spec((_D, _D)), _full_spec((1, _D)),
                  _full_spec((_D, _D)), _full_spec((1, _D))],
        out_specs=[_row_spec()],
        out_shape=[jax.ShapeDtypeStruct((_NP, _D), jnp.float32)],
    )(t0, t1, h2p, dinv, bg2, z, z1, z0, w2a, w2b, w2c, b2, w3, b3, wo, bo)[0]


def kernel(x, edge_index, gx, W1, b1, Wdr, bdr, Wg1, bg1, Wg2, bg2, W2, b2,
           W3, b3, Wout, bout):
    xp = jnp.pad(x, ((0, _NP - _N), (0, 0)))
    gxp = jnp.pad(gx, ((0, _NP - _N), (0, 0)))
    pad_e = _NW * _EPT - _E
    srcp = jnp.concatenate(
        [edge_index[0], jnp.full((pad_e,), _N, jnp.int32)]
    ).reshape(_TOTCH, _CHUNK)
    dstp = jnp.concatenate(
        [edge_index[1], jnp.full((pad_e,), _PAD_DST, jnp.int32)]
    ).reshape(_TOTCH, _CHUNK)

    w1a, w1b = W1[:_D], W1[_D:]
    w2a, w2b, w2c = W2[:_D], W2[_D:2 * _D], W2[2 * _D:]
    wo = jnp.pad(Wout, ((0, 0), (0, _D - Wout.shape[1])))
    bo = jnp.pad(bout, ((0, _D - bout.shape[0]),)).reshape(1, _D)
    b1r = b1.reshape(1, _D)
    bdrr = bdr.reshape(1, _D)
    bg1r = bg1.reshape(1, _D)
    bg2r = bg2.reshape(1, _D)
    b2r = b2.reshape(1, _D)
    b3r = b3.reshape(1, _D)

    degp = _sc_degree(dstp)
    z, z0, h1 = _tc0(xp, gxp, w1a, w1b, b1r, Wdr, bdrr, Wg1)
    h1p, dinv = _tc1(degp[0].reshape(_NP, 1), degp[1].reshape(_NP, 1), h1)
    s = _sc_scatter(h1p, srcp, dstp)
    z1, h2p = _tc2(s[0], s[1], h1p, dinv, bg1r, Wg2)
    t = _sc_scatter(h2p, srcp, dstp)
    o = _tc3(t[0], t[1], h2p, dinv, bg2r, z, z1, z0, w2a, w2b, w2c, b2r,
             W3, b3r, wo, bo)
    return o[:_N, :40]


# speedup vs baseline: 2.1324x; 2.1324x over previous
import functools

import jax
import jax.numpy as jnp
from jax import lax
from jax.experimental import pallas as pl
from jax.experimental.pallas import tpu as pltpu
from jax.experimental.pallas import tpu_sc as plsc

_N = 10000
_D = 128
_E = 320000
_NC, _NS = 2, 16
_NW = _NC * _NS
_CHUNK = 128
_TOTCH = _E // _CHUNK
_NP = 10240
_RPS = _NP // _NS


def _sc_mesh():
    return plsc.VectorSubcoreMesh(core_axis_name="c", subcore_axis_name="s")


def _sc_degree(ei):
    @functools.partial(
        pl.kernel,
        mesh=_sc_mesh(),
        out_type=jax.ShapeDtypeStruct((_NC, _NP), jnp.float32),
        scratch_types=[
            pltpu.VMEM((_CHUNK,), jnp.int32),
            pltpu.VMEM((_CHUNK,), jnp.float32),
            pltpu.VMEM_SHARED((_NP,), jnp.float32),
        ],
    )
    def k(ei_hbm, out_hbm, dst_v, ones_v, deg_sh):
        cid = lax.axis_index("c")
        sid = lax.axis_index("s")
        wid = sid * _NC + cid
        start = wid * _TOTCH // _NW
        nch = (wid + 1) * _TOTCH // _NW - start

        ones16 = jnp.ones((16,), jnp.float32)
        zeros16 = jnp.zeros((16,), jnp.float32)

        def _z(r, c):
            ones_v[pl.ds(r * 16, 16)] = zeros16
            return c

        lax.fori_loop(0, _CHUNK // 16, _z, 0)
        base = sid * _RPS
        for t in range(_RPS // _CHUNK):
            pltpu.sync_copy(ones_v, deg_sh.at[pl.ds(base + t * _CHUNK, _CHUNK)])

        def _o(r, c):
            ones_v[pl.ds(r * 16, 16)] = ones16
            return c

        lax.fori_loop(0, _CHUNK // 16, _o, 0)
        plsc.subcore_barrier()

        def _step(j, c):
            pltpu.sync_copy(ei_hbm.at[1, pl.ds((start + j) * _CHUNK, _CHUNK)],
                            dst_v)
            pltpu.sync_copy(ones_v, deg_sh.at[dst_v], add=True)
            return c

        lax.fori_loop(0, nch, _step, 0)
        plsc.subcore_barrier()
        pltpu.sync_copy(deg_sh.at[pl.ds(base, _RPS)],
                        out_hbm.at[cid, pl.ds(base, _RPS)])

    return k(ei)


def _sc_scatter(h_tab, ei):
    @functools.partial(
        pl.kernel,
        mesh=_sc_mesh(),
        out_type=jax.ShapeDtypeStruct((_NC, _NP, _D), jnp.float32),
        scratch_types=[
            pltpu.VMEM((_CHUNK,), jnp.int32),
            pltpu.VMEM((_CHUNK,), jnp.int32),
            pltpu.VMEM((_CHUNK, _D), jnp.float32),
            pltpu.SemaphoreType.DMA,
            pltpu.VMEM_SHARED((_NP, _D), jnp.float32),
        ],
    )
    def k(h_hbm, ei_hbm, out_hbm, src_v, dst_v, rows_v, sem, acc_sh):
        cid = lax.axis_index("c")
        sid = lax.axis_index("s")
        half = _TOTCH // 2
        cstart = cid * half
        start = cstart + sid * half // _NS
        nch = cstart + (sid + 1) * half // _NS - start

        zeros16 = jnp.zeros((16,), jnp.float32)

        def _z(r, c):
            for q in range(_D // 16):
                rows_v[r, pl.ds(q * 16, 16)] = zeros16
            return c

        lax.fori_loop(0, _CHUNK, _z, 0)

        base = sid * _RPS
        for t in range(_RPS // _CHUNK):
            pltpu.sync_copy(rows_v, acc_sh.at[pl.ds(base + t * _CHUNK, _CHUNK)])
        plsc.subcore_barrier()

        def _step(j, c):
            pltpu.sync_copy(ei_hbm.at[0, pl.ds((start + j) * _CHUNK, _CHUNK)],
                            src_v)
            pltpu.sync_copy(ei_hbm.at[1, pl.ds((start + j) * _CHUNK, _CHUNK)],
                            dst_v)
            pltpu.async_copy(h_hbm.at[src_v], rows_v, sem).wait()
            pltpu.sync_copy(rows_v, acc_sh.at[dst_v], add=True)
            return c

        lax.fori_loop(0, nch, _step, 0)
        plsc.subcore_barrier()
        pltpu.sync_copy(acc_sh.at[pl.ds(base, _RPS)],
                        out_hbm.at[cid, pl.ds(base, _RPS)])

    return k(h_tab, ei)


_R = 2000
_G = _N // _R


def _dot(a, b):
    return jnp.dot(a, b, preferred_element_type=jnp.float32)


def _row_spec(width=_D):
    return pl.BlockSpec((_R, width), lambda i: (i, 0))


def _full_spec(shape):
    nd = len(shape)
    return pl.BlockSpec(shape, lambda i: (0,) * nd)


def _tc0(xp, gxp, w1a, w1b, b1, wdr, bdr, wg1):
    def body(x_r, gx_r, w1a_r, w1b_r, b1_r, wdr_r, bdr_r, wg1_r,
             z_r, z0_r, h1_r):
        gg = gx_r[...]
        z = jnp.maximum(
            _dot(x_r[...], w1a_r[...]) + _dot(gg, w1b_r[...]) + b1_r[...], 0.0)
        z_r[...] = z
        z0_r[...] = _dot(z, wdr_r[...]) + bdr_r[...]
        h1_r[...] = _dot(z + gg, wg1_r[...])

    out = jax.ShapeDtypeStruct((_N, _D), jnp.float32)
    return pl.pallas_call(
        body,
        grid=(_G,),
        in_specs=[_row_spec(), _row_spec(),
                  _full_spec((_D, _D)), _full_spec((_D, _D)),
                  _full_spec((1, _D)),
                  _full_spec((_D, _D)), _full_spec((1, _D)),
                  _full_spec((_D, _D))],
        out_specs=[_row_spec(), _row_spec(), _row_spec()],
        out_shape=[out, out, out],
    )(xp, gxp, w1a, w1b, b1, wdr, bdr, wg1)


def _tc1(d0, d1, h1):
    def body(d0_r, d1_r, h1_r, hp_r, dinv_r):
        deg = d0_r[...] + d1_r[...] + 1.0
        dinv = lax.rsqrt(deg)
        dinv_r[...] = dinv
        hp_r[...] = dinv * h1_r[...]

    return pl.pallas_call(
        body,
        grid=(_G,),
        in_specs=[_row_spec(1), _row_spec(1), _row_spec()],
        out_specs=[_row_spec(), _row_spec(1)],
        out_shape=[jax.ShapeDtypeStruct((_N, _D), jnp.float32),
                   jax.ShapeDtypeStruct((_N, 1), jnp.float32)],
    )(d0, d1, h1)


def _tc2(s0, s1, hp, dinv, bg1, wg2):
    def body(s0_r, s1_r, hp_r, dinv_r, bg1_r, wg2_r, z1_r, h2p_r):
        dv = dinv_r[...]
        z1 = jnp.maximum(dv * (s0_r[...] + s1_r[...] + hp_r[...]) + bg1_r[...],
                         0.0)
        z1_r[...] = z1
        h2p_r[...] = dv * _dot(z1, wg2_r[...])

    out = jax.ShapeDtypeStruct((_N, _D), jnp.float32)
    return pl.pallas_call(
        body,
        grid=(_G,),
        in_specs=[_row_spec(), _row_spec(), _row_spec(), _row_spec(1),
                  _full_spec((1, _D)), _full_spec((_D, _D))],
        out_specs=[_row_spec(), _row_spec()],
        out_shape=[out, out],
    )(s0, s1, hp, dinv, bg1, wg2)


def _tc3(t0, t1, h2p, dinv, bg2, z, z1, z0, w2a, w2b, w2c, b2, w3, b3, wo, bo):
    def body(t0_r, t1_r, h2p_r, dinv_r, bg2_r, z_r, z1_r, z0_r,
             w2a_r, w2b_r, w2c_r, b2_r, w3_r, b3_r, wo_r, bo_r, o_r):
        dv = dinv_r[...]
        z2 = jnp.maximum(
            dv * (t0_r[...] + t1_r[...] + h2p_r[...]) + bg2_r[...], 0.0)
        z3 = jnp.maximum(
            _dot(z_r[...], w2a_r[...]) + _dot(z1_r[...], w2b_r[...])
            + _dot(z2, w2c_r[...]) + b2_r[...], 0.0)
        z4 = jnp.maximum(_dot(z3 + z0_r[...], w3_r[...]) + b3_r[...], 0.0)
        o_r[...] = _dot(z4, wo_r[...]) + bo_r[...]

    no = 40
    return pl.pallas_call(
        body,
        grid=(_G,),
        in_specs=[_row_spec(), _row_spec(), _row_spec(), _row_spec(1),
                  _full_spec((1, _D)),
                  _row_spec(), _row_spec(), _row_spec(),
                  _full_spec((_D, _D)), _full_spec((_D, _D)),
                  _full_spec((_D, _D)), _full_spec((1, _D)),
                  _full_spec((_D, _D)), _full_spec((1, _D)),
                  _full_spec((_D, no)), _full_spec((1, no))],
        out_specs=[_row_spec(no)],
        out_shape=[jax.ShapeDtypeStruct((_N, no), jnp.float32)],
    )(t0, t1, h2p, dinv, bg2, z, z1, z0, w2a, w2b, w2c, b2, w3, b3, wo, bo)[0]


def kernel(x, edge_index, gx, W1, b1, Wdr, bdr, Wg1, bg1, Wg2, bg2, W2, b2,
           W3, b3, Wout, bout):
    w1a, w1b = W1[:_D], W1[_D:]
    w2a, w2b, w2c = W2[:_D], W2[_D:2 * _D], W2[2 * _D:]
    b1r = b1.reshape(1, _D)
    bdrr = bdr.reshape(1, _D)
    bg1r = bg1.reshape(1, _D)
    bg2r = bg2.reshape(1, _D)
    b2r = b2.reshape(1, _D)
    b3r = b3.reshape(1, _D)
    bor = bout.reshape(1, -1)

    degp = _sc_degree(edge_index)
    z, z0, h1 = _tc0(x, gx, w1a, w1b, b1r, Wdr, bdrr, Wg1)
    h1p, dinv = _tc1(degp[0].reshape(_NP, 1), degp[1].reshape(_NP, 1), h1)
    s = _sc_scatter(h1p, edge_index)
    z1, h2p = _tc2(s[0], s[1], h1p, dinv, bg1r, Wg2)
    t = _sc_scatter(h2p, edge_index)
    o = _tc3(t[0], t[1], h2p, dinv, bg2r, z, z1, z0, w2a, w2b, w2c, b2r,
             W3, b3r, Wout, bor)
    return o


# speedup vs baseline: 2.5632x; 1.2020x over previous
import functools

import jax
import jax.numpy as jnp
from jax import lax
from jax.experimental import pallas as pl
from jax.experimental.pallas import tpu as pltpu
from jax.experimental.pallas import tpu_sc as plsc

_N = 10000
_D = 128
_E = 320000
_NC, _NS = 2, 16
_NW = _NC * _NS
_CHUNK = 128
_TOTCH = _E // _CHUNK
_NP = 10240
_RPS = _NP // _NS


def _sc_mesh():
    return plsc.VectorSubcoreMesh(core_axis_name="c", subcore_axis_name="s")


def _sc_degree(ei):
    @functools.partial(
        pl.kernel,
        mesh=_sc_mesh(),
        out_type=jax.ShapeDtypeStruct((_NC, _NP), jnp.float32),
        scratch_types=[
            pltpu.VMEM((_CHUNK,), jnp.int32),
            pltpu.VMEM((_CHUNK,), jnp.int32),
            pltpu.VMEM((_CHUNK,), jnp.float32),
            pltpu.SemaphoreType.DMA,
            pltpu.SemaphoreType.DMA,
            pltpu.VMEM_SHARED((_NP,), jnp.float32),
        ],
    )
    def k(ei_hbm, out_hbm, dst_a, dst_b, ones_v, sem_a, sem_b, deg_sh):
        cid = lax.axis_index("c")
        sid = lax.axis_index("s")
        wid = sid * _NC + cid
        start = wid * _TOTCH // _NW
        nch = (wid + 1) * _TOTCH // _NW - start

        ones16 = jnp.ones((16,), jnp.float32)
        zeros16 = jnp.zeros((16,), jnp.float32)

        def _z(r, c):
            ones_v[pl.ds(r * 16, 16)] = zeros16
            return c

        lax.fori_loop(0, _CHUNK // 16, _z, 0)
        base = sid * _RPS
        for t in range(_RPS // _CHUNK):
            pltpu.sync_copy(ones_v, deg_sh.at[pl.ds(base + t * _CHUNK, _CHUNK)])

        def _o(r, c):
            ones_v[pl.ds(r * 16, 16)] = ones16
            return c

        lax.fori_loop(0, _CHUNK // 16, _o, 0)
        plsc.subcore_barrier()

        def _idx(j, buf, sem):
            pltpu.async_copy(
                ei_hbm.at[1, pl.ds((start + j) * _CHUNK, _CHUNK)], buf, sem)

        def _wait(buf, sem):
            pltpu.make_async_copy(
                ei_hbm.at[1, pl.ds(0, _CHUNK)], buf, sem).wait()

        _idx(0, dst_a, sem_a)

        def _pair(t, c):
            j1 = 2 * t + 1
            _wait(dst_a, sem_a)

            @pl.when(j1 < nch)
            def _():
                _idx(j1, dst_b, sem_b)

            pltpu.sync_copy(ones_v, deg_sh.at[dst_a], add=True)

            @pl.when(j1 < nch)
            def _():
                _wait(dst_b, sem_b)

                @pl.when(j1 + 1 < nch)
                def _():
                    _idx(j1 + 1, dst_a, sem_a)

                pltpu.sync_copy(ones_v, deg_sh.at[dst_b], add=True)

            return c

        lax.fori_loop(0, (nch + 1) // 2, _pair, 0)
        plsc.subcore_barrier()
        pltpu.sync_copy(deg_sh.at[pl.ds(base, _RPS)],
                        out_hbm.at[cid, pl.ds(base, _RPS)])

    return k(ei)


def _sc_scatter(h_tab, ei):
    @functools.partial(
        pl.kernel,
        mesh=_sc_mesh(),
        out_type=jax.ShapeDtypeStruct((_NC, _NP, _D), jnp.float32),
        scratch_types=[
            pltpu.VMEM((_CHUNK,), jnp.int32),
            pltpu.VMEM((_CHUNK,), jnp.int32),
            pltpu.VMEM((_CHUNK,), jnp.int32),
            pltpu.VMEM((_CHUNK,), jnp.int32),
            pltpu.VMEM((_CHUNK, _D), jnp.float32),
            pltpu.VMEM((_CHUNK, _D), jnp.float32),
            pltpu.SemaphoreType.DMA,
            pltpu.SemaphoreType.DMA,
            pltpu.VMEM_SHARED((_NP, _D), jnp.float32),
        ],
    )
    def k(h_hbm, ei_hbm, out_hbm, src_a, dst_a, src_b, dst_b, rows_a, rows_b,
          sem_a, sem_b, acc_sh):
        cid = lax.axis_index("c")
        sid = lax.axis_index("s")
        half = _TOTCH // 2
        cstart = cid * half
        start = cstart + sid * half // _NS
        nch = cstart + (sid + 1) * half // _NS - start

        zeros16 = jnp.zeros((16,), jnp.float32)

        def _z(r, c):
            for q in range(_D // 16):
                rows_a[r, pl.ds(q * 16, 16)] = zeros16
            return c

        lax.fori_loop(0, _CHUNK, _z, 0)

        base = sid * _RPS
        for t in range(_RPS // _CHUNK):
            pltpu.sync_copy(rows_a, acc_sh.at[pl.ds(base + t * _CHUNK, _CHUNK)])
        plsc.subcore_barrier()

        def _ldidx(j, sv, dv):
            pltpu.sync_copy(ei_hbm.at[0, pl.ds((start + j) * _CHUNK, _CHUNK)],
                            sv)
            pltpu.sync_copy(ei_hbm.at[1, pl.ds((start + j) * _CHUNK, _CHUNK)],
                            dv)

        def _wait_gather(sv, rv, sem):
            pltpu.make_async_copy(h_hbm.at[sv], rv, sem).wait()

        _ldidx(0, src_a, dst_a)
        pltpu.async_copy(h_hbm.at[src_a], rows_a, sem_a)

        def _pair(t, c):
            j1 = 2 * t + 1
            _wait_gather(src_a, rows_a, sem_a)

            @pl.when(j1 < nch)
            def _():
                _ldidx(j1, src_b, dst_b)
                pltpu.async_copy(h_hbm.at[src_b], rows_b, sem_b)

            pltpu.sync_copy(rows_a, acc_sh.at[dst_a], add=True)

            @pl.when(j1 < nch)
            def _():
                _wait_gather(src_b, rows_b, sem_b)

                @pl.when(j1 + 1 < nch)
                def _():
                    _ldidx(j1 + 1, src_a, dst_a)
                    pltpu.async_copy(h_hbm.at[src_a], rows_a, sem_a)

                pltpu.sync_copy(rows_b, acc_sh.at[dst_b], add=True)

            return c

        lax.fori_loop(0, (nch + 1) // 2, _pair, 0)
        plsc.subcore_barrier()
        pltpu.sync_copy(acc_sh.at[pl.ds(base, _RPS)],
                        out_hbm.at[cid, pl.ds(base, _RPS)])

    return k(h_tab, ei)


_R = 2000
_G = _N // _R


def _dot(a, b):
    return jnp.dot(a, b, preferred_element_type=jnp.float32)


def _row_spec(width=_D):
    return pl.BlockSpec((_R, width), lambda i: (i, 0))


def _full_spec(shape):
    nd = len(shape)
    return pl.BlockSpec(shape, lambda i: (0,) * nd)


def _tc0(xp, gxp, w1a, w1b, b1, wdr, bdr, wg1):
    def body(x_r, gx_r, w1a_r, w1b_r, b1_r, wdr_r, bdr_r, wg1_r,
             z_r, z0_r, h1_r):
        gg = gx_r[...]
        z = jnp.maximum(
            _dot(x_r[...], w1a_r[...]) + _dot(gg, w1b_r[...]) + b1_r[...], 0.0)
        z_r[...] = z
        z0_r[...] = _dot(z, wdr_r[...]) + bdr_r[...]
        h1_r[...] = _dot(z + gg, wg1_r[...])

    out = jax.ShapeDtypeStruct((_N, _D), jnp.float32)
    return pl.pallas_call(
        body,
        grid=(_G,),
        in_specs=[_row_spec(), _row_spec(),
                  _full_spec((_D, _D)), _full_spec((_D, _D)),
                  _full_spec((1, _D)),
                  _full_spec((_D, _D)), _full_spec((1, _D)),
                  _full_spec((_D, _D))],
        out_specs=[_row_spec(), _row_spec(), _row_spec()],
        out_shape=[out, out, out],
    )(xp, gxp, w1a, w1b, b1, wdr, bdr, wg1)


def _tc1(d0, d1, h1):
    def body(d0_r, d1_r, h1_r, hp_r, dinv_r):
        deg = d0_r[...] + d1_r[...] + 1.0
        dinv = lax.rsqrt(deg)
        dinv_r[...] = dinv
        hp_r[...] = dinv * h1_r[...]

    return pl.pallas_call(
        body,
        grid=(_G,),
        in_specs=[_row_spec(1), _row_spec(1), _row_spec()],
        out_specs=[_row_spec(), _row_spec(1)],
        out_shape=[jax.ShapeDtypeStruct((_N, _D), jnp.float32),
                   jax.ShapeDtypeStruct((_N, 1), jnp.float32)],
    )(d0, d1, h1)


def _tc2(s0, s1, hp, dinv, bg1, wg2):
    def body(s0_r, s1_r, hp_r, dinv_r, bg1_r, wg2_r, z1_r, h2p_r):
        dv = dinv_r[...]
        z1 = jnp.maximum(dv * (s0_r[...] + s1_r[...] + hp_r[...]) + bg1_r[...],
                         0.0)
        z1_r[...] = z1
        h2p_r[...] = dv * _dot(z1, wg2_r[...])

    out = jax.ShapeDtypeStruct((_N, _D), jnp.float32)
    return pl.pallas_call(
        body,
        grid=(_G,),
        in_specs=[_row_spec(), _row_spec(), _row_spec(), _row_spec(1),
                  _full_spec((1, _D)), _full_spec((_D, _D))],
        out_specs=[_row_spec(), _row_spec()],
        out_shape=[out, out],
    )(s0, s1, hp, dinv, bg1, wg2)


def _tc3(t0, t1, h2p, dinv, bg2, z, z1, z0, w2a, w2b, w2c, b2, w3, b3, wo, bo):
    def body(t0_r, t1_r, h2p_r, dinv_r, bg2_r, z_r, z1_r, z0_r,
             w2a_r, w2b_r, w2c_r, b2_r, w3_r, b3_r, wo_r, bo_r, o_r):
        dv = dinv_r[...]
        z2 = jnp.maximum(
            dv * (t0_r[...] + t1_r[...] + h2p_r[...]) + bg2_r[...], 0.0)
        z3 = jnp.maximum(
            _dot(z_r[...], w2a_r[...]) + _dot(z1_r[...], w2b_r[...])
            + _dot(z2, w2c_r[...]) + b2_r[...], 0.0)
        z4 = jnp.maximum(_dot(z3 + z0_r[...], w3_r[...]) + b3_r[...], 0.0)
        o_r[...] = _dot(z4, wo_r[...]) + bo_r[...]

    no = 40
    return pl.pallas_call(
        body,
        grid=(_G,),
        in_specs=[_row_spec(), _row_spec(), _row_spec(), _row_spec(1),
                  _full_spec((1, _D)),
                  _row_spec(), _row_spec(), _row_spec(),
                  _full_spec((_D, _D)), _full_spec((_D, _D)),
                  _full_spec((_D, _D)), _full_spec((1, _D)),
                  _full_spec((_D, _D)), _full_spec((1, _D)),
                  _full_spec((_D, no)), _full_spec((1, no))],
        out_specs=[_row_spec(no)],
        out_shape=[jax.ShapeDtypeStruct((_N, no), jnp.float32)],
    )(t0, t1, h2p, dinv, bg2, z, z1, z0, w2a, w2b, w2c, b2, w3, b3, wo, bo)[0]


def kernel(x, edge_index, gx, W1, b1, Wdr, bdr, Wg1, bg1, Wg2, bg2, W2, b2,
           W3, b3, Wout, bout):
    w1a, w1b = W1[:_D], W1[_D:]
    w2a, w2b, w2c = W2[:_D], W2[_D:2 * _D], W2[2 * _D:]
    b1r = b1.reshape(1, _D)
    bdrr = bdr.reshape(1, _D)
    bg1r = bg1.reshape(1, _D)
    bg2r = bg2.reshape(1, _D)
    b2r = b2.reshape(1, _D)
    b3r = b3.reshape(1, _D)
    bor = bout.reshape(1, -1)

    degp = _sc_degree(edge_index)
    z, z0, h1 = _tc0(x, gx, w1a, w1b, b1r, Wdr, bdrr, Wg1)
    h1p, dinv = _tc1(degp[0].reshape(_NP, 1), degp[1].reshape(_NP, 1), h1)
    s = _sc_scatter(h1p, edge_index)
    z1, h2p = _tc2(s[0], s[1], h1p, dinv, bg1r, Wg2)
    t = _sc_scatter(h2p, edge_index)
    o = _tc3(t[0], t[1], h2p, dinv, bg2r, z, z1, z0, w2a, w2b, w2c, b2r,
             W3, b3r, Wout, bor)
    return o


# speedup vs baseline: 3.4852x; 1.3597x over previous
import functools

import jax
import jax.numpy as jnp
from jax import lax
from jax.experimental import pallas as pl
from jax.experimental.pallas import tpu as pltpu
from jax.experimental.pallas import tpu_sc as plsc

_N = 10000
_D = 128
_E = 320000
_NC, _NS = 2, 16
_NW = _NC * _NS
_CHUNK = 128
_TOTCH = _E // _CHUNK
_NP = 10240
_RPS = _NP // _NS


def _sc_mesh():
    return plsc.VectorSubcoreMesh(core_axis_name="c", subcore_axis_name="s")


def _sc_degree(ei):
    @functools.partial(
        pl.kernel,
        mesh=_sc_mesh(),
        out_type=jax.ShapeDtypeStruct((_NC, _NP), jnp.float32),
        scratch_types=[
            pltpu.VMEM((_CHUNK,), jnp.int32),
            pltpu.VMEM((_CHUNK,), jnp.int32),
            pltpu.VMEM((_CHUNK,), jnp.float32),
            pltpu.SemaphoreType.DMA,
            pltpu.SemaphoreType.DMA,
            pltpu.VMEM_SHARED((_NP,), jnp.float32),
        ],
    )
    def k(ei_hbm, out_hbm, dst_a, dst_b, ones_v, sem_a, sem_b, deg_sh):
        cid = lax.axis_index("c")
        sid = lax.axis_index("s")
        wid = sid * _NC + cid
        start = wid * _TOTCH // _NW
        nch = (wid + 1) * _TOTCH // _NW - start

        ones16 = jnp.ones((16,), jnp.float32)
        zeros16 = jnp.zeros((16,), jnp.float32)

        def _z(r, c):
            ones_v[pl.ds(r * 16, 16)] = zeros16
            return c

        lax.fori_loop(0, _CHUNK // 16, _z, 0)
        base = sid * _RPS
        for t in range(_RPS // _CHUNK):
            pltpu.sync_copy(ones_v, deg_sh.at[pl.ds(base + t * _CHUNK, _CHUNK)])

        def _o(r, c):
            ones_v[pl.ds(r * 16, 16)] = ones16
            return c

        lax.fori_loop(0, _CHUNK // 16, _o, 0)
        plsc.subcore_barrier()

        def _idx(j, buf, sem):
            pltpu.async_copy(
                ei_hbm.at[1, pl.ds((start + j) * _CHUNK, _CHUNK)], buf, sem)

        def _wait(buf, sem):
            pltpu.make_async_copy(
                ei_hbm.at[1, pl.ds(0, _CHUNK)], buf, sem).wait()

        _idx(0, dst_a, sem_a)

        def _pair(t, c):
            j1 = 2 * t + 1
            _wait(dst_a, sem_a)

            @pl.when(j1 < nch)
            def _():
                _idx(j1, dst_b, sem_b)

            pltpu.sync_copy(ones_v, deg_sh.at[dst_a], add=True)

            @pl.when(j1 < nch)
            def _():
                _wait(dst_b, sem_b)

                @pl.when(j1 + 1 < nch)
                def _():
                    _idx(j1 + 1, dst_a, sem_a)

                pltpu.sync_copy(ones_v, deg_sh.at[dst_b], add=True)

            return c

        lax.fori_loop(0, (nch + 1) // 2, _pair, 0)
        plsc.subcore_barrier()
        pltpu.sync_copy(deg_sh.at[pl.ds(base, _RPS)],
                        out_hbm.at[cid, pl.ds(base, _RPS)])

    return k(ei)


def _sc_scatter(h_tab, ei):
    @functools.partial(
        pl.kernel,
        mesh=_sc_mesh(),
        out_type=jax.ShapeDtypeStruct((_NC, _NP, _D), jnp.float32),
        scratch_types=[
            pltpu.VMEM((_CHUNK,), jnp.int32),
            pltpu.VMEM((_CHUNK,), jnp.int32),
            pltpu.VMEM((_CHUNK,), jnp.int32),
            pltpu.VMEM((_CHUNK,), jnp.int32),
            pltpu.VMEM((_CHUNK,), jnp.int32),
            pltpu.VMEM((_CHUNK,), jnp.int32),
            pltpu.VMEM((_CHUNK, _D), jnp.float32),
            pltpu.VMEM((_CHUNK, _D), jnp.float32),
            pltpu.SemaphoreType.DMA,
            pltpu.SemaphoreType.DMA,
            pltpu.SemaphoreType.DMA,
            pltpu.SemaphoreType.DMA,
            pltpu.SemaphoreType.DMA,
            pltpu.VMEM_SHARED((_NP, _D), jnp.float32),
        ],
    )
    def k(h_hbm, ei_hbm, out_hbm, src_a, dst_a, src_b, dst_b, src_c, dst_c,
          rows_a, rows_b, sem_ga, sem_gb, sem_ia, sem_ib, sem_ic, acc_sh):
        cid = lax.axis_index("c")
        sid = lax.axis_index("s")
        half = _TOTCH // 2
        cstart = cid * half
        start = cstart + sid * half // _NS
        nch = cstart + (sid + 1) * half // _NS - start

        zeros16 = jnp.zeros((16,), jnp.float32)

        def _z(r, c):
            for q in range(_D // 16):
                rows_a[r, pl.ds(q * 16, 16)] = zeros16
            return c

        lax.fori_loop(0, _CHUNK, _z, 0)

        base = sid * _RPS
        for t in range(_RPS // _CHUNK):
            pltpu.sync_copy(rows_a, acc_sh.at[pl.ds(base + t * _CHUNK, _CHUNK)])
        plsc.subcore_barrier()

        rows = [rows_a, rows_b]
        semg = [sem_ga, sem_gb]
        idx = [(src_a, dst_a, sem_ia), (src_b, dst_b, sem_ib),
               (src_c, dst_c, sem_ic)]

        def _idx_async(j, buf):
            sv, dv, sem = buf
            pltpu.async_copy(
                ei_hbm.at[0, pl.ds((start + j) * _CHUNK, _CHUNK)], sv, sem)
            pltpu.async_copy(
                ei_hbm.at[1, pl.ds((start + j) * _CHUNK, _CHUNK)], dv, sem)

        def _idx_wait(buf):
            sv, dv, sem = buf
            pltpu.make_async_copy(
                ei_hbm.at[0, pl.ds(0, _CHUNK)], sv, sem).wait()
            pltpu.make_async_copy(
                ei_hbm.at[1, pl.ds(0, _CHUNK)], dv, sem).wait()

        def _gather_wait(sv, rv, sem):
            pltpu.make_async_copy(h_hbm.at[sv], rv, sem).wait()

        _idx_async(0, idx[0])
        _idx_async(1, idx[1])
        _idx_wait(idx[0])
        pltpu.async_copy(h_hbm.at[src_a], rows_a, sem_ga)

        def _stage(u, t):
            j = 6 * t + u
            rX, rY = rows[u % 2], rows[(u + 1) % 2]
            iJ, iJ1, iJ2 = idx[u % 3], idx[(u + 1) % 3], idx[(u + 2) % 3]

            @pl.when(j < nch)
            def _():
                _gather_wait(iJ[0], rX, semg[u % 2])

                @pl.when(j + 2 < nch)
                def _():
                    _idx_async(j + 2, iJ2)

                @pl.when(j + 1 < nch)
                def _():
                    _idx_wait(iJ1)
                    pltpu.async_copy(h_hbm.at[iJ1[0]], rY, semg[(u + 1) % 2])

                pltpu.sync_copy(rX, acc_sh.at[iJ[1]], add=True)

        def _six(t, c):
            for u in range(6):
                _stage(u, t)
            return c

        lax.fori_loop(0, (nch + 5) // 6, _six, 0)
        plsc.subcore_barrier()
        pltpu.sync_copy(acc_sh.at[pl.ds(base, _RPS)],
                        out_hbm.at[cid, pl.ds(base, _RPS)])

    return k(h_tab, ei)


_R = 2000
_G = _N // _R


def _dot(a, b):
    return jnp.dot(a, b, preferred_element_type=jnp.float32)


def _row_spec(width=_D):
    return pl.BlockSpec((_R, width), lambda i: (i, 0))


def _full_spec(shape):
    nd = len(shape)
    return pl.BlockSpec(shape, lambda i: (0,) * nd)


def _tc0(xp, gxp, w1a, w1b, b1, wdr, bdr, wg1):
    def body(x_r, gx_r, w1a_r, w1b_r, b1_r, wdr_r, bdr_r, wg1_r,
             z_r, z0_r, h1_r):
        gg = gx_r[...]
        z = jnp.maximum(
            _dot(x_r[...], w1a_r[...]) + _dot(gg, w1b_r[...]) + b1_r[...], 0.0)
        z_r[...] = z
        z0_r[...] = _dot(z, wdr_r[...]) + bdr_r[...]
        h1_r[...] = _dot(z + gg, wg1_r[...])

    out = jax.ShapeDtypeStruct((_N, _D), jnp.float32)
    return pl.pallas_call(
        body,
        grid=(_G,),
        in_specs=[_row_spec(), _row_spec(),
                  _full_spec((_D, _D)), _full_spec((_D, _D)),
                  _full_spec((1, _D)),
                  _full_spec((_D, _D)), _full_spec((1, _D)),
                  _full_spec((_D, _D))],
        out_specs=[_row_spec(), _row_spec(), _row_spec()],
        out_shape=[out, out, out],
    )(xp, gxp, w1a, w1b, b1, wdr, bdr, wg1)


def _tc1(d0, d1, h1):
    def body(d0_r, d1_r, h1_r, hp_r, dinv_r):
        deg = d0_r[...] + d1_r[...] + 1.0
        dinv = lax.rsqrt(deg)
        dinv_r[...] = dinv
        hp_r[...] = dinv * h1_r[...]

    return pl.pallas_call(
        body,
        grid=(_G,),
        in_specs=[_row_spec(1), _row_spec(1), _row_spec()],
        out_specs=[_row_spec(), _row_spec(1)],
        out_shape=[jax.ShapeDtypeStruct((_N, _D), jnp.float32),
                   jax.ShapeDtypeStruct((_N, 1), jnp.float32)],
    )(d0, d1, h1)


def _tc2(s0, s1, hp, dinv, bg1, wg2):
    def body(s0_r, s1_r, hp_r, dinv_r, bg1_r, wg2_r, z1_r, h2p_r):
        dv = dinv_r[...]
        z1 = jnp.maximum(dv * (s0_r[...] + s1_r[...] + hp_r[...]) + bg1_r[...],
                         0.0)
        z1_r[...] = z1
        h2p_r[...] = dv * _dot(z1, wg2_r[...])

    out = jax.ShapeDtypeStruct((_N, _D), jnp.float32)
    return pl.pallas_call(
        body,
        grid=(_G,),
        in_specs=[_row_spec(), _row_spec(), _row_spec(), _row_spec(1),
                  _full_spec((1, _D)), _full_spec((_D, _D))],
        out_specs=[_row_spec(), _row_spec()],
        out_shape=[out, out],
    )(s0, s1, hp, dinv, bg1, wg2)


def _tc3(t0, t1, h2p, dinv, bg2, z, z1, z0, w2a, w2b, w2c, b2, w3, b3, wo, bo):
    def body(t0_r, t1_r, h2p_r, dinv_r, bg2_r, z_r, z1_r, z0_r,
             w2a_r, w2b_r, w2c_r, b2_r, w3_r, b3_r, wo_r, bo_r, o_r):
        dv = dinv_r[...]
        z2 = jnp.maximum(
            dv * (t0_r[...] + t1_r[...] + h2p_r[...]) + bg2_r[...], 0.0)
        z3 = jnp.maximum(
            _dot(z_r[...], w2a_r[...]) + _dot(z1_r[...], w2b_r[...])
            + _dot(z2, w2c_r[...]) + b2_r[...], 0.0)
        z4 = jnp.maximum(_dot(z3 + z0_r[...], w3_r[...]) + b3_r[...], 0.0)
        o_r[...] = _dot(z4, wo_r[...]) + bo_r[...]

    no = 40
    return pl.pallas_call(
        body,
        grid=(_G,),
        in_specs=[_row_spec(), _row_spec(), _row_spec(), _row_spec(1),
                  _full_spec((1, _D)),
                  _row_spec(), _row_spec(), _row_spec(),
                  _full_spec((_D, _D)), _full_spec((_D, _D)),
                  _full_spec((_D, _D)), _full_spec((1, _D)),
                  _full_spec((_D, _D)), _full_spec((1, _D)),
                  _full_spec((_D, no)), _full_spec((1, no))],
        out_specs=[_row_spec(no)],
        out_shape=[jax.ShapeDtypeStruct((_N, no), jnp.float32)],
    )(t0, t1, h2p, dinv, bg2, z, z1, z0, w2a, w2b, w2c, b2, w3, b3, wo, bo)[0]


def kernel(x, edge_index, gx, W1, b1, Wdr, bdr, Wg1, bg1, Wg2, bg2, W2, b2,
           W3, b3, Wout, bout):
    w1a, w1b = W1[:_D], W1[_D:]
    w2a, w2b, w2c = W2[:_D], W2[_D:2 * _D], W2[2 * _D:]
    b1r = b1.reshape(1, _D)
    bdrr = bdr.reshape(1, _D)
    bg1r = bg1.reshape(1, _D)
    bg2r = bg2.reshape(1, _D)
    b2r = b2.reshape(1, _D)
    b3r = b3.reshape(1, _D)
    bor = bout.reshape(1, -1)

    degp = _sc_degree(edge_index)
    z, z0, h1 = _tc0(x, gx, w1a, w1b, b1r, Wdr, bdrr, Wg1)
    h1p, dinv = _tc1(degp[0].reshape(_NP, 1), degp[1].reshape(_NP, 1), h1)
    s = _sc_scatter(h1p, edge_index)
    z1, h2p = _tc2(s[0], s[1], h1p, dinv, bg1r, Wg2)
    t = _sc_scatter(h2p, edge_index)
    o = _tc3(t[0], t[1], h2p, dinv, bg2r, z, z1, z0, w2a, w2b, w2c, b2r,
             W3, b3r, Wout, bor)
    return o


# speedup vs baseline: 3.7227x; 1.0681x over previous
import functools

import jax
import jax.numpy as jnp
from jax import lax
from jax.experimental import pallas as pl
from jax.experimental.pallas import tpu as pltpu
from jax.experimental.pallas import tpu_sc as plsc

_N = 10000
_D = 128
_E = 320000
_NC, _NS = 2, 16
_NW = _NC * _NS
_CHUNK = 128
_TOTCH = _E // _CHUNK
_NP = 10240
_RPS = _NP // _NS
_NPS = 10112
_SRPS = _NPS // _NS


def _sc_mesh():
    return plsc.VectorSubcoreMesh(core_axis_name="c", subcore_axis_name="s")


def _sc_degree(ei):
    @functools.partial(
        pl.kernel,
        mesh=_sc_mesh(),
        out_type=jax.ShapeDtypeStruct((_NC, _NP), jnp.float32),
        scratch_types=[
            pltpu.VMEM((_CHUNK,), jnp.int32),
            pltpu.VMEM((_CHUNK,), jnp.int32),
            pltpu.VMEM((_CHUNK,), jnp.float32),
            pltpu.SemaphoreType.DMA,
            pltpu.SemaphoreType.DMA,
            pltpu.VMEM_SHARED((_NP,), jnp.float32),
        ],
    )
    def k(ei_hbm, out_hbm, dst_a, dst_b, ones_v, sem_a, sem_b, deg_sh):
        cid = lax.axis_index("c")
        sid = lax.axis_index("s")
        wid = sid * _NC + cid
        start = wid * _TOTCH // _NW
        nch = (wid + 1) * _TOTCH // _NW - start

        ones16 = jnp.ones((16,), jnp.float32)
        zeros16 = jnp.zeros((16,), jnp.float32)

        def _z(r, c):
            ones_v[pl.ds(r * 16, 16)] = zeros16
            return c

        lax.fori_loop(0, _CHUNK // 16, _z, 0)
        base = sid * _RPS
        for t in range(_RPS // _CHUNK):
            pltpu.sync_copy(ones_v, deg_sh.at[pl.ds(base + t * _CHUNK, _CHUNK)])

        def _o(r, c):
            ones_v[pl.ds(r * 16, 16)] = ones16
            return c

        lax.fori_loop(0, _CHUNK // 16, _o, 0)
        plsc.subcore_barrier()

        def _idx(j, buf, sem):
            pltpu.async_copy(
                ei_hbm.at[1, pl.ds((start + j) * _CHUNK, _CHUNK)], buf, sem)

        def _wait(buf, sem):
            pltpu.make_async_copy(
                ei_hbm.at[1, pl.ds(0, _CHUNK)], buf, sem).wait()

        _idx(0, dst_a, sem_a)

        def _pair(t, c):
            j1 = 2 * t + 1
            _wait(dst_a, sem_a)

            @pl.when(j1 < nch)
            def _():
                _idx(j1, dst_b, sem_b)

            pltpu.sync_copy(ones_v, deg_sh.at[dst_a], add=True)

            @pl.when(j1 < nch)
            def _():
                _wait(dst_b, sem_b)

                @pl.when(j1 + 1 < nch)
                def _():
                    _idx(j1 + 1, dst_a, sem_a)

                pltpu.sync_copy(ones_v, deg_sh.at[dst_b], add=True)

            return c

        lax.fori_loop(0, (nch + 1) // 2, _pair, 0)
        plsc.subcore_barrier()
        pltpu.sync_copy(deg_sh.at[pl.ds(base, _RPS)],
                        out_hbm.at[cid, pl.ds(base, _RPS)])

    return k(ei)


def _sc_scatter(h_tab, ei):
    @functools.partial(
        pl.kernel,
        mesh=_sc_mesh(),
        out_type=jax.ShapeDtypeStruct((_NC, _NPS, _D), jnp.float32),
        scratch_types=[
            pltpu.VMEM((2, _CHUNK), jnp.int32),
            pltpu.VMEM((2, _CHUNK), jnp.int32),
            pltpu.VMEM((2, _CHUNK), jnp.int32),
            pltpu.VMEM((_CHUNK, _D), jnp.float32),
            pltpu.VMEM((_CHUNK, _D), jnp.float32),
            pltpu.VMEM((_CHUNK, _D), jnp.float32),
            pltpu.SemaphoreType.DMA,
            pltpu.SemaphoreType.DMA,
            pltpu.SemaphoreType.DMA,
            pltpu.SemaphoreType.DMA,
            pltpu.SemaphoreType.DMA,
            pltpu.SemaphoreType.DMA,
            pltpu.VMEM_SHARED((_NPS, _D), jnp.float32),
        ],
    )
    def k(h_hbm, ei_hbm, out_hbm, ia, ib, ic, ra, rb, rc,
          sia, sib, sic, sga, sgb, sgc, acc_sh):
        cid = lax.axis_index("c")
        sid = lax.axis_index("s")
        half = _TOTCH // 2
        cstart = cid * half
        start = cstart + sid * half // _NS
        nch = cstart + (sid + 1) * half // _NS - start

        zeros16 = jnp.zeros((16,), jnp.float32)

        def _z(r, c):
            for q in range(_D // 16):
                ra[r, pl.ds(q * 16, 16)] = zeros16
            return c

        lax.fori_loop(0, _CHUNK, _z, 0)

        base = sid * _SRPS
        for t in range(4):
            pltpu.sync_copy(ra, acc_sh.at[pl.ds(base + t * _CHUNK, _CHUNK)])
        pltpu.sync_copy(ra.at[pl.ds(0, _SRPS - 4 * _CHUNK)],
                        acc_sh.at[pl.ds(base + 4 * _CHUNK,
                                        _SRPS - 4 * _CHUNK)])
        plsc.subcore_barrier()

        idx = [(ia, sia), (ib, sib), (ic, sic)]
        rows = [(ra, sga), (rb, sgb), (rc, sgc)]

        def _idx_issue(j, u):
            buf, sem = idx[u]
            pltpu.async_copy(
                ei_hbm.at[:, pl.ds((start + j) * _CHUNK, _CHUNK)], buf, sem)

        def _idx_wait(u):
            buf, sem = idx[u]
            pltpu.make_async_copy(
                ei_hbm.at[:, pl.ds(0, _CHUNK)], buf, sem).wait()

        def _g_issue(u):
            pltpu.async_copy(h_hbm.at[idx[u][0].at[0]], rows[u][0],
                             rows[u][1])

        def _g_wait(u):
            pltpu.make_async_copy(h_hbm.at[idx[u][0].at[0]], rows[u][0],
                                  rows[u][1]).wait()

        _idx_issue(0, 0)
        _idx_issue(1, 1)
        _idx_issue(2, 2)
        _idx_wait(0)
        _g_issue(0)
        _idx_wait(1)
        _g_issue(1)

        def _stage(u, t):
            j = 3 * t + u

            @pl.when(j < nch)
            def _():
                _g_wait(u)

                @pl.when(j + 2 < nch)
                def _():
                    _idx_wait((u + 2) % 3)
                    _g_issue((u + 2) % 3)

                pltpu.sync_copy(rows[u][0], acc_sh.at[idx[u][0].at[1]],
                                add=True)

                @pl.when(j + 3 < nch)
                def _():
                    _idx_issue(j + 3, u)

        def _tri(t, c):
            for u in range(3):
                _stage(u, t)
            return c

        lax.fori_loop(0, (nch + 2) // 3, _tri, 0)
        plsc.subcore_barrier()
        pltpu.sync_copy(acc_sh.at[pl.ds(base, _SRPS)],
                        out_hbm.at[cid, pl.ds(base, _SRPS)])

    return k(h_tab, ei)


_R = 2000
_G = _N // _R


def _dot(a, b):
    return jnp.dot(a, b, preferred_element_type=jnp.float32)


def _row_spec(width=_D):
    return pl.BlockSpec((_R, width), lambda i: (i, 0))


def _full_spec(shape):
    nd = len(shape)
    return pl.BlockSpec(shape, lambda i: (0,) * nd)


def _tc0(xp, gxp, w1a, w1b, b1, wdr, bdr, wg1):
    def body(x_r, gx_r, w1a_r, w1b_r, b1_r, wdr_r, bdr_r, wg1_r,
             z_r, z0_r, h1_r):
        gg = gx_r[...]
        z = jnp.maximum(
            _dot(x_r[...], w1a_r[...]) + _dot(gg, w1b_r[...]) + b1_r[...], 0.0)
        z_r[...] = z
        z0_r[...] = _dot(z, wdr_r[...]) + bdr_r[...]
        h1_r[...] = _dot(z + gg, wg1_r[...])

    out = jax.ShapeDtypeStruct((_N, _D), jnp.float32)
    return pl.pallas_call(
        body,
        grid=(_G,),
        in_specs=[_row_spec(), _row_spec(),
                  _full_spec((_D, _D)), _full_spec((_D, _D)),
                  _full_spec((1, _D)),
                  _full_spec((_D, _D)), _full_spec((1, _D)),
                  _full_spec((_D, _D))],
        out_specs=[_row_spec(), _row_spec(), _row_spec()],
        out_shape=[out, out, out],
    )(xp, gxp, w1a, w1b, b1, wdr, bdr, wg1)


def _tc1(d0, d1, h1):
    def body(d0_r, d1_r, h1_r, hp_r, dinv_r):
        deg = d0_r[...] + d1_r[...] + 1.0
        dinv = lax.rsqrt(deg)
        dinv_r[...] = dinv
        hp_r[...] = dinv * h1_r[...]

    return pl.pallas_call(
        body,
        grid=(_G,),
        in_specs=[_row_spec(1), _row_spec(1), _row_spec()],
        out_specs=[_row_spec(), _row_spec(1)],
        out_shape=[jax.ShapeDtypeStruct((_N, _D), jnp.float32),
                   jax.ShapeDtypeStruct((_N, 1), jnp.float32)],
    )(d0, d1, h1)


def _tc2(s0, s1, hp, dinv, bg1, wg2):
    def body(s0_r, s1_r, hp_r, dinv_r, bg1_r, wg2_r, z1_r, h2p_r):
        dv = dinv_r[...]
        z1 = jnp.maximum(dv * (s0_r[...] + s1_r[...] + hp_r[...]) + bg1_r[...],
                         0.0)
        z1_r[...] = z1
        h2p_r[...] = dv * _dot(z1, wg2_r[...])

    out = jax.ShapeDtypeStruct((_N, _D), jnp.float32)
    return pl.pallas_call(
        body,
        grid=(_G,),
        in_specs=[_row_spec(), _row_spec(), _row_spec(), _row_spec(1),
                  _full_spec((1, _D)), _full_spec((_D, _D))],
        out_specs=[_row_spec(), _row_spec()],
        out_shape=[out, out],
    )(s0, s1, hp, dinv, bg1, wg2)


def _tc3(t0, t1, h2p, dinv, bg2, z, z1, z0, w2a, w2b, w2c, b2, w3, b3, wo, bo):
    def body(t0_r, t1_r, h2p_r, dinv_r, bg2_r, z_r, z1_r, z0_r,
             w2a_r, w2b_r, w2c_r, b2_r, w3_r, b3_r, wo_r, bo_r, o_r):
        dv = dinv_r[...]
        z2 = jnp.maximum(
            dv * (t0_r[...] + t1_r[...] + h2p_r[...]) + bg2_r[...], 0.0)
        z3 = jnp.maximum(
            _dot(z_r[...], w2a_r[...]) + _dot(z1_r[...], w2b_r[...])
            + _dot(z2, w2c_r[...]) + b2_r[...], 0.0)
        z4 = jnp.maximum(_dot(z3 + z0_r[...], w3_r[...]) + b3_r[...], 0.0)
        o_r[...] = _dot(z4, wo_r[...]) + bo_r[...]

    no = 40
    return pl.pallas_call(
        body,
        grid=(_G,),
        in_specs=[_row_spec(), _row_spec(), _row_spec(), _row_spec(1),
                  _full_spec((1, _D)),
                  _row_spec(), _row_spec(), _row_spec(),
                  _full_spec((_D, _D)), _full_spec((_D, _D)),
                  _full_spec((_D, _D)), _full_spec((1, _D)),
                  _full_spec((_D, _D)), _full_spec((1, _D)),
                  _full_spec((_D, no)), _full_spec((1, no))],
        out_specs=[_row_spec(no)],
        out_shape=[jax.ShapeDtypeStruct((_N, no), jnp.float32)],
    )(t0, t1, h2p, dinv, bg2, z, z1, z0, w2a, w2b, w2c, b2, w3, b3, wo, bo)[0]


def kernel(x, edge_index, gx, W1, b1, Wdr, bdr, Wg1, bg1, Wg2, bg2, W2, b2,
           W3, b3, Wout, bout):
    w1a, w1b = W1[:_D], W1[_D:]
    w2a, w2b, w2c = W2[:_D], W2[_D:2 * _D], W2[2 * _D:]
    b1r = b1.reshape(1, _D)
    bdrr = bdr.reshape(1, _D)
    bg1r = bg1.reshape(1, _D)
    bg2r = bg2.reshape(1, _D)
    b2r = b2.reshape(1, _D)
    b3r = b3.reshape(1, _D)
    bor = bout.reshape(1, -1)

    degp = _sc_degree(edge_index)
    z, z0, h1 = _tc0(x, gx, w1a, w1b, b1r, Wdr, bdrr, Wg1)
    h1p, dinv = _tc1(degp[0].reshape(_NP, 1), degp[1].reshape(_NP, 1), h1)
    s = _sc_scatter(h1p, edge_index)
    z1, h2p = _tc2(s[0], s[1], h1p, dinv, bg1r, Wg2)
    t = _sc_scatter(h2p, edge_index)
    o = _tc3(t[0], t[1], h2p, dinv, bg2r, z, z1, z0, w2a, w2b, w2c, b2r,
             W3, b3r, Wout, bor)
    return o
